# baseline (speedup 1.0000x reference)
import jax
import jax.numpy as jnp
from jax import lax
from jax.experimental import pallas as pl
from jax.experimental.pallas import tpu as pltpu
from jax.experimental.pallas import tpu_sc as plsc

_NC, _NS = 2, 16
_NW = _NC * _NS
_T, _D, _B = 8192, 1024, 4
_TPW = _T // _NW
_RB = 32
_NB = _TPW // _RB
_CH = _RB * _D


def _sc_body(x_hbm, e_hbm, o_hbm, ebuf, buf, sem):
    w = lax.axis_index("s") * _NC + lax.axis_index("c")
    t0 = w * _TPW
    for blk in range(_NB):
        e_off = (t0 + blk * _RB) * _D
        pltpu.sync_copy(e_hbm.at[pl.ds(e_off, _CH)], ebuf)
        for b in range(_B):
            x_off = b * (_T * _D) + e_off
            pltpu.sync_copy(x_hbm.at[pl.ds(x_off, _CH)], buf)

            @plsc.parallel_loop(0, _CH, step=16, unroll=8)
            def _add(i):
                s = pl.ds(i, 16)
                buf[s] = buf[s] + ebuf[s]

            pltpu.sync_copy(buf, o_hbm.at[pl.ds(x_off, _CH)])


def kernel(x, embedding):
    Bx, Tx, Dx = x.shape
    xf = x.reshape(Bx * Tx * Dx)
    ef = embedding.reshape(Tx * Dx)
    run = pl.kernel(
        _sc_body,
        out_type=jax.ShapeDtypeStruct((Bx * Tx * Dx,), x.dtype),
        mesh=plsc.VectorSubcoreMesh(
            core_axis_name="c", subcore_axis_name="s",
            num_cores=_NC, num_subcores=_NS,
        ),
        scratch_types=[
            pltpu.VMEM((_CH,), jnp.float32),
            pltpu.VMEM((_CH,), jnp.float32),
            pltpu.SemaphoreType.DMA,
        ],
    )
    return run(xf, ef).reshape(Bx, Tx, Dx)


# speedup vs baseline: 1.1792x; 1.1792x over previous
import jax
import jax.numpy as jnp
from jax import lax
from jax.experimental import pallas as pl
from jax.experimental.pallas import tpu as pltpu
from jax.experimental.pallas import tpu_sc as plsc

_NC, _NS = 2, 16
_NW = _NC * _NS
_T, _D, _B = 8192, 1024, 4
_TPW = _T // _NW
_RB = 16
_NB = _TPW // _RB
_CH = _RB * _D
_NBUF = 3
_STEPS = _NB * _B


def _sc_body(x_hbm, e_hbm, o_hbm, xb0, xb1, xb2, eb,
             i0, i1, i2, o0, o1, o2):
    bufs = (xb0, xb1, xb2)
    isems = (i0, i1, i2)
    osems = (o0, o1, o2)
    w = lax.axis_index("s") * _NC + lax.axis_index("c")
    t0 = w * _TPW

    def x_slice(step):
        blk, b = divmod(step, _B)
        off = b * (_T * _D) + (t0 + blk * _RB) * _D
        return pl.ds(off, _CH)

    pltpu.async_copy(x_hbm.at[x_slice(0)], bufs[0], isems[0])

    for step in range(_STEPS):
        blk, b = divmod(step, _B)
        k = step % _NBUF
        if b == 0:
            e_off = (t0 + blk * _RB) * _D
            pltpu.sync_copy(e_hbm.at[pl.ds(e_off, _CH)], eb)
        nxt = step + 1
        if nxt < _STEPS:
            kn = nxt % _NBUF
            if nxt >= _NBUF:
                pltpu.make_async_copy(
                    bufs[kn], o_hbm.at[x_slice(nxt - _NBUF)], osems[kn]
                ).wait()
            pltpu.async_copy(x_hbm.at[x_slice(nxt)], bufs[kn], isems[kn])
        pltpu.make_async_copy(x_hbm.at[x_slice(step)], bufs[k], isems[k]).wait()

        buf = bufs[k]

        @plsc.parallel_loop(0, _CH, step=16, unroll=8)
        def _add(i):
            s = pl.ds(i, 16)
            buf[s] = buf[s] + eb[s]

        pltpu.async_copy(bufs[k], o_hbm.at[x_slice(step)], osems[k])

    for step in range(_STEPS - _NBUF, _STEPS):
        k = step % _NBUF
        pltpu.make_async_copy(
            bufs[k], o_hbm.at[x_slice(step)], osems[k]
        ).wait()


def kernel(x, embedding):
    Bx, Tx, Dx = x.shape
    xf = x.reshape(Bx * Tx * Dx)
    ef = embedding.reshape(Tx * Dx)
    run = pl.kernel(
        _sc_body,
        out_type=jax.ShapeDtypeStruct((Bx * Tx * Dx,), x.dtype),
        mesh=plsc.VectorSubcoreMesh(
            core_axis_name="c", subcore_axis_name="s",
            num_cores=_NC, num_subcores=_NS,
        ),
        scratch_types=[
            pltpu.VMEM((_CH,), jnp.float32),
            pltpu.VMEM((_CH,), jnp.float32),
            pltpu.VMEM((_CH,), jnp.float32),
            pltpu.VMEM((_CH,), jnp.float32),
            pltpu.SemaphoreType.DMA,
            pltpu.SemaphoreType.DMA,
            pltpu.SemaphoreType.DMA,
            pltpu.SemaphoreType.DMA,
            pltpu.SemaphoreType.DMA,
            pltpu.SemaphoreType.DMA,
        ],
    )
    return run(xf, ef).reshape(Bx, Tx, Dx)


# speedup vs baseline: 3.4248x; 2.9044x over previous
import jax
import jax.numpy as jnp
from jax import lax
from jax.experimental import pallas as pl
from jax.experimental.pallas import tpu as pltpu
from jax.experimental.pallas import tpu_sc as plsc

_NC, _NS = 2, 16
_NW = _NC * _NS
_T, _D, _B = 8192, 1024, 4
_TPW = _T // _NW
_RB = 8
_NB = _TPW // _RB


def _sc_body(x_hbm, e_hbm, o_hbm,
             xa0, xa1, xa2, xa3, ea,
             xb0, xb1, xb2, xb3, eb,
             isa, esa, osa, isb, esb, osb):
    xbufs = ((xa0, xa1, xa2, xa3), (xb0, xb1, xb2, xb3))
    ebufs = (ea, eb)
    isems = (isa, isb)
    esems = (esa, esb)
    osems = (osa, osb)
    w = lax.axis_index("s") * _NC + lax.axis_index("c")
    t0 = w * _TPW

    def x_rows(g, j):
        return pl.ds(j * _T + t0 + g * _RB, _RB)

    def e_rows(g):
        return pl.ds(t0 + g * _RB, _RB)

    def start_in(g, p):
        for j in range(_B):
            pltpu.async_copy(x_hbm.at[x_rows(g, j)], xbufs[p][j], isems[p])
        pltpu.async_copy(e_hbm.at[e_rows(g)], ebufs[p], esems[p])

    def wait_in(g, p):
        for j in range(_B):
            pltpu.make_async_copy(
                x_hbm.at[x_rows(g, j)], xbufs[p][j], isems[p]).wait()
        pltpu.make_async_copy(e_hbm.at[e_rows(g)], ebufs[p], esems[p]).wait()

    def wait_out(g, p):
        for j in range(_B):
            pltpu.make_async_copy(
                xbufs[p][j], o_hbm.at[x_rows(g, j)], osems[p]).wait()

    def add_block(p, j):
        buf = xbufs[p][j]
        ebf = ebufs[p]
        for r in range(_RB):
            @plsc.parallel_loop(0, _D, step=16, unroll=8)
            def _add(i):
                s = pl.ds(i, 16)
                buf[r, s] = buf[r, s] + ebf[r, s]

    start_in(0, 0)

    def body(gg, carry):
        for par in range(2):
            g = 2 * gg + par
            q = 1 - par
            wait_in(g, par)
            add_block(par, 0)
            pltpu.async_copy(xbufs[par][0], o_hbm.at[x_rows(g, 0)], osems[par])
            @pl.when(g > 0)
            def _():
                wait_out(g - 1, q)
            @pl.when(g + 1 < _NB)
            def _():
                start_in(g + 1, q)
            for j in range(1, _B):
                add_block(par, j)
                pltpu.async_copy(
                    xbufs[par][j], o_hbm.at[x_rows(g, j)], osems[par])
        return carry

    lax.fori_loop(0, _NB // 2, body, 0)
    wait_out(_NB - 1, 1)


def kernel(x, embedding):
    Bx, Tx, Dx = x.shape
    x2 = x.reshape(Bx * Tx, Dx)
    run = pl.kernel(
        _sc_body,
        out_type=jax.ShapeDtypeStruct((Bx * Tx, Dx), x.dtype),
        mesh=plsc.VectorSubcoreMesh(
            core_axis_name="c", subcore_axis_name="s",
            num_cores=_NC, num_subcores=_NS,
        ),
        scratch_types=(
            [pltpu.VMEM((_RB, _D), jnp.float32)] * 5
            + [pltpu.VMEM((_RB, _D), jnp.float32)] * 5
            + [pltpu.SemaphoreType.DMA] * 6
        ),
        compiler_params=pltpu.CompilerParams(use_tc_tiling_on_sc=True),
    )
    return run(x2, embedding).reshape(Bx, Tx, Dx)


# speedup vs baseline: 5.1346x; 1.4993x over previous
import jax
import jax.numpy as jnp
from jax.experimental import pallas as pl
from jax.experimental.pallas import tpu as pltpu


_TB = 256


def _add_body(x_ref, e_ref, o_ref):
    o_ref[...] = x_ref[...] + e_ref[...][None, :, :]


def kernel(x, embedding):
    B, T, D = x.shape
    grid = (T // _TB,)
    return pl.pallas_call(
        _add_body,
        grid=grid,
        in_specs=[
            pl.BlockSpec((B, _TB, D), lambda i: (0, i, 0)),
            pl.BlockSpec((_TB, D), lambda i: (i, 0)),
        ],
        out_specs=pl.BlockSpec((B, _TB, D), lambda i: (0, i, 0)),
        out_shape=jax.ShapeDtypeStruct((B, T, D), x.dtype),
        compiler_params=pltpu.CompilerParams(
            dimension_semantics=("arbitrary",),
        ),
    )(x, embedding)
